# baseline (speedup 1.0000x reference)
import functools

import jax
import jax.numpy as jnp
from jax import lax
from jax.experimental import pallas as pl
from jax.experimental.pallas import tpu as pltpu
from jax.experimental.pallas import tpu_sc as plsc

_BATCH = 16384
_NC = 1
_NS = 16
_LANES = 16
_NW = _NC * _NS
_CHUNK = _BATCH // _NW


def _lane0_broadcast(vec):
    return lax.gather(
        vec, jnp.zeros((_LANES, 1), jnp.int32),
        dimension_numbers=lax.GatherDimensionNumbers(
            offset_dims=(), collapsed_slice_dims=(0,), start_index_map=(0,)),
        slice_sizes=(1,),
        mode=lax.GatherScatterMode.PROMISE_IN_BOUNDS)


def _sc_body(tau_hbm, idx_hbm, theta_hbm, mu_hbm, out_hbm,
             idx_v, th_v, mu_v, tau_v, out_v, tau_sem, gather_sem):
    wid = lax.axis_index("s") * _NC + lax.axis_index("c")
    base = wid * _CHUNK
    idx_v[...] = jnp.zeros((_LANES,), jnp.int32)
    idx_cp = pltpu.async_copy(idx_hbm, idx_v.at[pl.ds(0, 1)], gather_sem)
    tau_cp = pltpu.async_copy(tau_hbm.at[pl.ds(base, _CHUNK)], tau_v, tau_sem)
    idx_cp.wait()
    th_cp = pltpu.async_copy(theta_hbm.at[idx_v], th_v, gather_sem)
    mu_cp = pltpu.async_copy(mu_hbm.at[idx_v], mu_v, gather_sem)
    th_cp.wait()
    mu_cp.wait()
    th = _lane0_broadcast(th_v[...])
    m = _lane0_broadcast(mu_v[...])
    tau_cp.wait()
    half = _CHUNK // 2
    out_cps = []
    for h in range(2):
        for j in range(h * half // _LANES, (h + 1) * half // _LANES):
            x = tau_v[pl.ds(j * _LANES, _LANES)]
            z = th + m * x
            out_v[pl.ds(j * _LANES, _LANES)] = 1.0 / (1.0 + jnp.exp(-z))
        out_cps.append(pltpu.async_copy(
            out_v.at[pl.ds(h * half, half)],
            out_hbm.at[pl.ds(base + h * half, half)], tau_sem))
    for cp in out_cps:
        cp.wait()


@functools.partial(
    pl.kernel,
    mesh=plsc.VectorSubcoreMesh(core_axis_name="c", subcore_axis_name="s",
                                num_cores=_NC),
    out_type=jax.ShapeDtypeStruct((_BATCH,), jnp.float32),
    scratch_types=[
        pltpu.VMEM((_LANES,), jnp.int32),
        pltpu.VMEM((_LANES,), jnp.float32),
        pltpu.VMEM((_LANES,), jnp.float32),
        pltpu.VMEM((_CHUNK,), jnp.float32),
        pltpu.VMEM((_CHUNK,), jnp.float32),
        pltpu.SemaphoreType.DMA,
        pltpu.SemaphoreType.DMA,
    ],
    compiler_params=pltpu.CompilerParams(skip_device_barrier=True),
)
def _sc_kernel(*refs):
    _sc_body(*refs)


def kernel(tau, inputs, theta, mu):
    return _sc_kernel(tau, inputs, theta, mu)


# speedup vs baseline: 1.0661x; 1.0661x over previous
import functools

import jax
import jax.numpy as jnp
from jax import lax
from jax.experimental import pallas as pl
from jax.experimental.pallas import tpu as pltpu
from jax.experimental.pallas import tpu_sc as plsc

_BATCH = 16384
_NC = 1
_NS = 16
_LANES = 16
_NW = _NC * _NS
_CHUNK = _BATCH // _NW


def _lane0_broadcast(vec):
    return lax.gather(
        vec, jnp.zeros((_LANES, 1), jnp.int32),
        dimension_numbers=lax.GatherDimensionNumbers(
            offset_dims=(), collapsed_slice_dims=(0,), start_index_map=(0,)),
        slice_sizes=(1,),
        mode=lax.GatherScatterMode.PROMISE_IN_BOUNDS)


def _sc_body(tau_hbm, idx_hbm, theta_hbm, mu_hbm, out_hbm,
             idx_v, th_v, mu_v, tau_v, out_v, tau_sem, gather_sem):
    wid = lax.axis_index("s") * _NC + lax.axis_index("c")
    base = wid * _CHUNK
    idx_v[...] = jnp.zeros((_LANES,), jnp.int32)
    idx_cp = pltpu.async_copy(idx_hbm, idx_v.at[pl.ds(0, 1)], gather_sem)
    tau_cp = pltpu.async_copy(tau_hbm.at[pl.ds(base, _CHUNK)], tau_v, tau_sem)
    idx_cp.wait()
    th_cp = pltpu.async_copy(theta_hbm.at[idx_v], th_v, gather_sem)
    mu_cp = pltpu.async_copy(mu_hbm.at[idx_v], mu_v, gather_sem)
    th_cp.wait()
    mu_cp.wait()
    th = _lane0_broadcast(th_v[...])
    m = _lane0_broadcast(mu_v[...])
    tau_cp.wait()

    @plsc.parallel_loop(0, _CHUNK, _LANES, unroll=4)
    def _compute(off):
        x = tau_v[pl.ds(off, _LANES)]
        z = th + m * x
        out_v[pl.ds(off, _LANES)] = 1.0 / (1.0 + jnp.exp(-z))

    pltpu.sync_copy(out_v, out_hbm.at[pl.ds(base, _CHUNK)])


@functools.partial(
    pl.kernel,
    mesh=plsc.VectorSubcoreMesh(core_axis_name="c", subcore_axis_name="s",
                                num_cores=_NC),
    out_type=jax.ShapeDtypeStruct((_BATCH,), jnp.float32),
    scratch_types=[
        pltpu.VMEM((_LANES,), jnp.int32),
        pltpu.VMEM((_LANES,), jnp.float32),
        pltpu.VMEM((_LANES,), jnp.float32),
        pltpu.VMEM((_CHUNK,), jnp.float32),
        pltpu.VMEM((_CHUNK,), jnp.float32),
        pltpu.SemaphoreType.DMA,
        pltpu.SemaphoreType.DMA,
    ],
)
def _sc_kernel(*refs):
    _sc_body(*refs)


def kernel(tau, inputs, theta, mu):
    return _sc_kernel(tau, inputs, theta, mu)


# speedup vs baseline: 1.0751x; 1.0084x over previous
import functools

import jax
import jax.numpy as jnp
from jax import lax
from jax.experimental import pallas as pl
from jax.experimental.pallas import tpu as pltpu
from jax.experimental.pallas import tpu_sc as plsc

_BATCH = 16384
_NC = 1
_NS = 16
_LANES = 16
_NW = _NC * _NS
_CHUNK = _BATCH // _NW


def _lane0_broadcast(vec):
    return lax.gather(
        vec, jnp.zeros((_LANES, 1), jnp.int32),
        dimension_numbers=lax.GatherDimensionNumbers(
            offset_dims=(), collapsed_slice_dims=(0,), start_index_map=(0,)),
        slice_sizes=(1,),
        mode=lax.GatherScatterMode.PROMISE_IN_BOUNDS)


def _sc_body(tau_hbm, idx_hbm, theta_hbm, mu_hbm, out_hbm,
             idx_v, th_v, mu_v, tau_v, tau_sem, gather_sem):
    wid = lax.axis_index("s") * _NC + lax.axis_index("c")
    base = wid * _CHUNK
    idx_v[...] = jnp.zeros((_LANES,), jnp.int32)
    idx_cp = pltpu.async_copy(idx_hbm, idx_v.at[pl.ds(0, 1)], gather_sem)
    tau_cp = pltpu.async_copy(tau_hbm.at[pl.ds(base, _CHUNK)], tau_v, tau_sem)
    idx_cp.wait()
    th_cp = pltpu.async_copy(theta_hbm.at[idx_v], th_v, gather_sem)
    mu_cp = pltpu.async_copy(mu_hbm.at[idx_v], mu_v, gather_sem)
    th_cp.wait()
    mu_cp.wait()
    th = _lane0_broadcast(th_v[...])
    m = _lane0_broadcast(mu_v[...])
    tau_cp.wait()

    @plsc.parallel_loop(0, _CHUNK, _LANES, unroll=2)
    def _compute(off):
        x = tau_v[pl.ds(off, _LANES)]
        z = th + m * x
        tau_v[pl.ds(off, _LANES)] = 1.0 / (1.0 + jnp.exp(-z))

    pltpu.sync_copy(tau_v, out_hbm.at[pl.ds(base, _CHUNK)])


@functools.partial(
    pl.kernel,
    mesh=plsc.VectorSubcoreMesh(core_axis_name="c", subcore_axis_name="s",
                                num_cores=_NC),
    out_type=jax.ShapeDtypeStruct((_BATCH,), jnp.float32),
    scratch_types=[
        pltpu.VMEM((_LANES,), jnp.int32),
        pltpu.VMEM((_LANES,), jnp.float32),
        pltpu.VMEM((_LANES,), jnp.float32),
        pltpu.VMEM((_CHUNK,), jnp.float32),
        pltpu.SemaphoreType.DMA,
        pltpu.SemaphoreType.DMA,
    ],
)
def _sc_kernel(*refs):
    _sc_body(*refs)


def kernel(tau, inputs, theta, mu):
    return _sc_kernel(tau, inputs, theta, mu)
